# Initial kernel scaffold; baseline (speedup 1.0000x reference)
import jax
import jax.numpy as jnp
from jax.experimental import pallas as pl


def kernel(z, edge_label_index):
    raise NotImplementedError("write your pallas kernel here")


# speedup vs baseline: 1.5370x; 1.5370x over previous
import functools

import jax
import jax.numpy as jnp
from jax import lax
from jax.experimental import pallas as pl
from jax.experimental.pallas import tpu as pltpu
from jax.experimental.pallas import tpu_sc as plsc

D = 256
B = 160000
E = 2 * B
NC, NS = 2, 16
NW = NC * NS
ROWS_PER_W = E // NW
CHUNK = 80
NCHUNK = ROWS_PER_W // CHUNK

_mesh = plsc.VectorSubcoreMesh(core_axis_name="c", subcore_axis_name="s")


@functools.partial(
    pl.kernel,
    mesh=_mesh,
    out_type=jax.ShapeDtypeStruct((E, D), jnp.float32),
    scratch_types=[
        pltpu.VMEM((NCHUNK, CHUNK), jnp.int32),
        pltpu.VMEM((CHUNK, D), jnp.float32),
        pltpu.SemaphoreType.DMA,
    ],
)
def _gather(z_hbm, idx_hbm, out_hbm, idx_v, rows_v, sem):
    wid = lax.axis_index("s") * NC + lax.axis_index("c")
    pltpu.sync_copy(idx_hbm.at[wid], idx_v)
    base = wid * ROWS_PER_W

    def body(c, carry):
        pltpu.async_copy(z_hbm.at[idx_v.at[c]], rows_v, sem).wait()
        pltpu.sync_copy(rows_v, out_hbm.at[pl.ds(base + c * CHUNK, CHUNK)])
        return carry

    lax.fori_loop(0, NCHUNK, body, 0, unroll=False)


def kernel(z, edge_label_index):
    idx = edge_label_index.astype(jnp.int32).T.reshape(NW, NCHUNK, CHUNK)
    out = _gather(z, idx)
    return out.reshape(B, 2 * D)


# speedup vs baseline: 1.7417x; 1.1332x over previous
import functools

import jax
import jax.numpy as jnp
from jax import lax
from jax.experimental import pallas as pl
from jax.experimental.pallas import tpu as pltpu
from jax.experimental.pallas import tpu_sc as plsc

D = 256
B = 160000
E = 2 * B
NC, NS = 2, 16
NW = NC * NS
ROWS_PER_W = E // NW
CHUNK = 80
NCHUNK = ROWS_PER_W // CHUNK
NBUF = 5
PRE = NBUF - 2
GROUPS = NCHUNK // NBUF

_mesh = plsc.VectorSubcoreMesh(core_axis_name="c", subcore_axis_name="s")


@functools.partial(
    pl.kernel,
    mesh=_mesh,
    out_type=jax.ShapeDtypeStruct((E, D), jnp.float32),
    scratch_types=[
        pltpu.VMEM((NCHUNK, CHUNK), jnp.int32),
        pltpu.VMEM((NBUF, CHUNK, D), jnp.float32),
    ]
    + [pltpu.SemaphoreType.DMA] * (2 * NBUF),
)
def _gather(z_hbm, idx_hbm, out_hbm, idx_v, rows, *sems):
    gsem, wsem = sems[:NBUF], sems[NBUF:]
    wid = lax.axis_index("s") * NC + lax.axis_index("c")
    pltpu.sync_copy(idx_hbm.at[wid], idx_v)
    base = wid * ROWS_PER_W

    def fire_gather(c, b):
        pltpu.async_copy(z_hbm.at[idx_v.at[c]], rows.at[b], gsem[b])

    def wait_gather(c, b):
        pltpu.make_async_copy(z_hbm.at[idx_v.at[c]], rows.at[b], gsem[b]).wait()

    def out_slice(c):
        return out_hbm.at[pl.ds(base + c * CHUNK, CHUNK)]

    def fire_write(c, b):
        pltpu.async_copy(rows.at[b], out_slice(c), wsem[b])

    def wait_write(c, b):
        pltpu.make_async_copy(rows.at[b], out_slice(c), wsem[b]).wait()

    def step(c, b):
        wait_gather(c, b)
        fire_write(c, b)
        f = c + PRE
        if f < NCHUNK:
            bf = (b + PRE) % NBUF
            if c >= 2:
                wait_write(c - 2, bf)
            fire_gather(f, bf)

    for c in range(PRE):
        fire_gather(c, c)
    for b in range(NBUF):
        step(b, b)

    def mid_group(g, carry):
        for b in range(NBUF):
            c = g * NBUF + b
            wait_gather(c, b)
            fire_write(c, b)
            bf = (b + PRE) % NBUF
            wait_write(c - 2, bf)
            fire_gather(c + PRE, bf)
        return carry

    lax.fori_loop(1, GROUPS - 1, mid_group, 0, unroll=False)

    for b in range(NBUF):
        step((GROUPS - 1) * NBUF + b, b)
    for b in range(NBUF):
        wait_write((GROUPS - 1) * NBUF + b, b)


def kernel(z, edge_label_index):
    idx = edge_label_index.astype(jnp.int32).T.reshape(NW, NCHUNK, CHUNK)
    out = _gather(z, idx)
    return out.reshape(B, 2 * D)


# speedup vs baseline: 4.6325x; 2.6598x over previous
import functools

import jax
import jax.numpy as jnp
from jax import lax
from jax.experimental import pallas as pl
from jax.experimental.pallas import tpu as pltpu
from jax.experimental.pallas import tpu_sc as plsc

D = 256
B = 160000
NC, NS = 2, 16
NW = NC * NS
NHALF = NW // 2
ROWS_PER_W = B // NHALF
CHUNK = 80
NCHUNK = ROWS_PER_W // CHUNK
NBUF = 5
PRE = NBUF - 2
GROUPS = NCHUNK // NBUF

_mesh = plsc.VectorSubcoreMesh(core_axis_name="c", subcore_axis_name="s")


@functools.partial(
    pl.kernel,
    mesh=_mesh,
    out_type=jax.ShapeDtypeStruct((B, 2 * D), jnp.float32),
    scratch_types=[
        pltpu.VMEM((NCHUNK, CHUNK), jnp.int32),
        pltpu.VMEM((NBUF, CHUNK, D), jnp.float32),
    ]
    + [pltpu.SemaphoreType.DMA] * (2 * NBUF),
)
def _gather(z_hbm, idx_hbm, out_hbm, idx_v, rows, *sems):
    gsem, wsem = sems[:NBUF], sems[NBUF:]
    wid = lax.axis_index("s") * NC + lax.axis_index("c")
    half = wid // NHALF
    lane = wid % NHALF
    pltpu.sync_copy(idx_hbm.at[half, lane], idx_v)
    row0 = lane * ROWS_PER_W
    col0 = half * D

    def fire_gather(c, b):
        pltpu.async_copy(z_hbm.at[idx_v.at[c]], rows.at[b], gsem[b])

    def wait_gather(c, b):
        pltpu.make_async_copy(z_hbm.at[idx_v.at[c]], rows.at[b], gsem[b]).wait()

    def out_slice(c):
        return out_hbm.at[pl.ds(row0 + c * CHUNK, CHUNK), pl.ds(col0, D)]

    def fire_write(c, b):
        pltpu.async_copy(rows.at[b], out_slice(c), wsem[b])

    def wait_write(c, b):
        pltpu.make_async_copy(rows.at[b], out_slice(c), wsem[b]).wait()

    def step(c, b):
        wait_gather(c, b)
        fire_write(c, b)
        f = c + PRE
        if f < NCHUNK:
            bf = (b + PRE) % NBUF
            if c >= 2:
                wait_write(c - 2, bf)
            fire_gather(f, bf)

    for c in range(PRE):
        fire_gather(c, c)
    for b in range(NBUF):
        step(b, b)

    def mid_group(g, carry):
        for b in range(NBUF):
            c = g * NBUF + b
            wait_gather(c, b)
            fire_write(c, b)
            bf = (b + PRE) % NBUF
            wait_write(c - 2, bf)
            fire_gather(c + PRE, bf)
        return carry

    lax.fori_loop(1, GROUPS - 1, mid_group, 0, unroll=False)

    for b in range(NBUF):
        step((GROUPS - 1) * NBUF + b, b)
    for b in range(NBUF):
        wait_write((GROUPS - 1) * NBUF + b, b)


def kernel(z, edge_label_index):
    idx = edge_label_index.astype(jnp.int32).reshape(2, NHALF, NCHUNK, CHUNK)
    return _gather(z, idx)
